# baseline (speedup 1.0000x reference)
import jax
import jax.numpy as jnp
from jax import lax
from jax.experimental import pallas as pl
from jax.experimental.pallas import tpu as pltpu
from jax.experimental.pallas import tpu_sc as plsc

N = 10000
E = 320000
D = 128
NC = 2
NS = 16
NWORK = NC * NS
BA = 128
CH = 16
TOT_ROWS = 2560
RPW0 = 96
RPW1 = 64
OFF1 = NS * RPW0
HB = 64
HROWS = 160
HCH = 16
E_PAD = TOT_ROWS * BA
NP = 10240
RPT = NP // NS

_mesh = plsc.VectorSubcoreMesh(
    core_axis_name="c", subcore_axis_name="s", num_cores=NC, num_subcores=NS
)


def _hist_body(dst_rows, out, dst_v, hist):
    c = lax.axis_index("c")
    s = lax.axis_index("s")
    w = s * NC + c

    def zbody(i, carry):
        hist[pl.ds(i * 16, 16)] = jnp.zeros((16,), jnp.float32)
        return carry

    lax.fori_loop(0, NP // 16, zbody, 0)
    ones = jnp.ones((16,), jnp.float32)

    def chunk(ci, carry):
        pltpu.sync_copy(dst_rows.at[pl.ds(w * HROWS + ci * HCH, HCH)], dst_v)

        def body(j, carry2):
            def inner(kk, carry3):
                idx = dst_v[j, pl.ds(kk * 16, 16)]
                plsc.addupdate_scatter(hist, (idx,), ones)
                return carry3

            lax.fori_loop(0, HB // 16, inner, 0)
            return carry2

        lax.fori_loop(0, HCH, body, 0)
        return carry

    lax.fori_loop(0, HROWS // HCH, chunk, 0)
    pltpu.sync_copy(hist, out.at[w])


_hist_sc = pl.kernel(
    _hist_body,
    mesh=_mesh,
    out_type=jax.ShapeDtypeStruct((NWORK, NP), jnp.float32),
    scratch_types=[
        pltpu.VMEM((HCH, HB), jnp.int32),
        pltpu.VMEM((NP,), jnp.float32),
    ],
    compiler_params=pltpu.CompilerParams(needs_layout_passes=False),
)


def _agg_body(gat_rows, sct_rows, table_hbm, z_hbm, out, gat_c, sct_c,
              rows0, rows1, acc_sh, sem0, sem1):
    c = lax.axis_index("c")
    s = lax.axis_index("s")
    pltpu.sync_copy(z_hbm.at[pl.ds(s * RPT, RPT)], acc_sh.at[pl.ds(s * RPT, RPT)])
    plsc.subcore_barrier()

    def chunk_body_at(row_base, ci, carry):
        base = row_base + ci * CH
        pltpu.sync_copy(gat_rows.at[pl.ds(base, CH)], gat_c)
        pltpu.sync_copy(sct_rows.at[pl.ds(base, CH)], sct_c)
        pltpu.async_copy(table_hbm.at[gat_c.at[0]], rows0, sem0)

        def body(t, carry2):
            j0 = 2 * t
            pltpu.make_async_copy(table_hbm.at[gat_c.at[j0]], rows0, sem0).wait()
            pltpu.async_copy(table_hbm.at[gat_c.at[j0 + 1]], rows1, sem1)
            pltpu.sync_copy(rows0, acc_sh.at[sct_c.at[j0]], add=True)
            pltpu.make_async_copy(table_hbm.at[gat_c.at[j0 + 1]], rows1, sem1).wait()

            @pl.when(j0 + 2 < CH)
            def _():
                pltpu.async_copy(
                    table_hbm.at[gat_c.at[jnp.minimum(j0 + 2, CH - 1)]], rows0, sem0
                )

            pltpu.sync_copy(rows1, acc_sh.at[sct_c.at[j0 + 1]], add=True)
            return carry2

        lax.fori_loop(0, CH // 2, body, 0)
        return carry

    @pl.when(c == 0)
    def _():
        lax.fori_loop(
            0, RPW0 // CH, lambda ci, cr: chunk_body_at(s * RPW0, ci, cr), 0
        )

    @pl.when(c == 1)
    def _():
        lax.fori_loop(
            0, RPW1 // CH, lambda ci, cr: chunk_body_at(OFF1 + s * RPW1, ci, cr), 0
        )

    plsc.subcore_barrier()
    pltpu.sync_copy(acc_sh.at[pl.ds(s * RPT, RPT)], out.at[c, pl.ds(s * RPT, RPT)])


_agg_sc = pl.kernel(
    _agg_body,
    mesh=_mesh,
    out_type=jax.ShapeDtypeStruct((NC, NP, D), jnp.float32),
    scratch_types=[
        pltpu.VMEM((CH, BA), jnp.int32),
        pltpu.VMEM((CH, BA), jnp.int32),
        pltpu.VMEM((BA, D), jnp.float32),
        pltpu.VMEM((BA, D), jnp.float32),
        pltpu.VMEM_SHARED((NP, D), jnp.float32),
        pltpu.SemaphoreType.DMA,
        pltpu.SemaphoreType.DMA,
    ],
)


def _h_body(hist_ref, feat_ref, h_ref):
    deg = jnp.sum(hist_ref[...], axis=0)[:, None]
    norm = lax.rsqrt(jnp.maximum(deg, 1.0))
    h_ref[...] = feat_ref[...] * norm


def _out_body(hist_ref, p_ref, wc_ref, bc_ref, wl_ref, bl_ref, o_ref):
    deg = jnp.sum(hist_ref[...], axis=0)[:, None]
    norm = lax.rsqrt(jnp.maximum(deg, 1.0))
    a = (p_ref[0] + p_ref[1]) * norm
    h1 = jnp.dot(a, wc_ref[...], preferred_element_type=jnp.float32) + bc_ref[...]
    h1 = jnp.maximum(h1, 0.0)
    h2 = jnp.dot(h1, wl_ref[...], preferred_element_type=jnp.float32) + bl_ref[...]
    o_ref[...] = jnp.maximum(h2, 0.0)


_BN = 512


def kernel(feat, edge_index, W_conv, b_conv, W_lin, b_lin):
    src = edge_index[0].astype(jnp.int32)
    dst = edge_index[1].astype(jnp.int32)
    pad = E_PAD - E
    src_p = jnp.concatenate([src, jnp.zeros((pad,), jnp.int32)])
    dst_p = jnp.concatenate([dst, jnp.full((pad,), N, jnp.int32)])
    src_rows = src_p.reshape(-1, BA)
    dst_rows = dst_p.reshape(-1, BA)
    feat_p = jnp.concatenate([feat, jnp.zeros((NP - N, D), jnp.float32)])
    z128 = jnp.zeros((NP, D), jnp.float32)

    hist = _hist_sc(dst_p.reshape(-1, HB))

    h = pl.pallas_call(
        _h_body,
        grid=(NP // _BN,),
        in_specs=[
            pl.BlockSpec((NWORK, _BN), lambda i: (0, i)),
            pl.BlockSpec((_BN, D), lambda i: (i, 0)),
        ],
        out_specs=pl.BlockSpec((_BN, D), lambda i: (i, 0)),
        out_shape=jax.ShapeDtypeStruct((NP, D), jnp.float32),
    )(hist, feat_p)

    agg_p = _agg_sc(src_rows, dst_rows, h, z128)

    out = pl.pallas_call(
        _out_body,
        grid=(NP // _BN,),
        in_specs=[
            pl.BlockSpec((NWORK, _BN), lambda i: (0, i)),
            pl.BlockSpec((NC, _BN, D), lambda i: (0, i, 0)),
            pl.BlockSpec((D, D), lambda i: (0, 0)),
            pl.BlockSpec((1, D), lambda i: (0, 0)),
            pl.BlockSpec((D, D), lambda i: (0, 0)),
            pl.BlockSpec((1, D), lambda i: (0, 0)),
        ],
        out_specs=pl.BlockSpec((_BN, D), lambda i: (i, 0)),
        out_shape=jax.ShapeDtypeStruct((NP, D), jnp.float32),
    )(hist, agg_p, W_conv.T, b_conv.reshape(1, D), W_lin.T, b_lin.reshape(1, D))

    return out[:N]


# speedup vs baseline: 1.0447x; 1.0447x over previous
import jax
import jax.numpy as jnp
from jax import lax
from jax.experimental import pallas as pl
from jax.experimental.pallas import tpu as pltpu
from jax.experimental.pallas import tpu_sc as plsc

N = 10000
E = 320000
D = 128
NC = 2
NS = 16
NWORK = NC * NS
BA = 128
CH = 16
TOT_ROWS = 2560
RPW0 = 128
RPW1 = 32
OFF1 = NS * RPW0
HB = 64
HROWS = 160
HCH = 16
E_PAD = TOT_ROWS * BA
NP = 10240
RPT = NP // NS

_mesh = plsc.VectorSubcoreMesh(
    core_axis_name="c", subcore_axis_name="s", num_cores=NC, num_subcores=NS
)


def _hist_body(dst_rows, out, dst_v, hist):
    c = lax.axis_index("c")
    s = lax.axis_index("s")
    w = s * NC + c

    def zbody(i, carry):
        hist[pl.ds(i * 16, 16)] = jnp.zeros((16,), jnp.float32)
        return carry

    lax.fori_loop(0, NP // 16, zbody, 0)
    ones = jnp.ones((16,), jnp.float32)

    def chunk(ci, carry):
        pltpu.sync_copy(dst_rows.at[pl.ds(w * HROWS + ci * HCH, HCH)], dst_v)

        def body(j, carry2):
            def inner(kk, carry3):
                idx = dst_v[j, pl.ds(kk * 16, 16)]
                plsc.addupdate_scatter(hist, (idx,), ones)
                return carry3

            lax.fori_loop(0, HB // 16, inner, 0)
            return carry2

        lax.fori_loop(0, HCH, body, 0)
        return carry

    lax.fori_loop(0, HROWS // HCH, chunk, 0)
    pltpu.sync_copy(hist, out.at[w])


_hist_sc = pl.kernel(
    _hist_body,
    mesh=_mesh,
    out_type=jax.ShapeDtypeStruct((NWORK, NP), jnp.float32),
    scratch_types=[
        pltpu.VMEM((HCH, HB), jnp.int32),
        pltpu.VMEM((NP,), jnp.float32),
    ],
    compiler_params=pltpu.CompilerParams(needs_layout_passes=False),
)


def _agg_body(gat_rows, sct_rows, table_hbm, z_hbm, out, gat_c, sct_c,
              rows0, rows1, acc_sh, sem0, sem1):
    c = lax.axis_index("c")
    s = lax.axis_index("s")
    pltpu.sync_copy(z_hbm.at[pl.ds(s * RPT, RPT)], acc_sh.at[pl.ds(s * RPT, RPT)])
    plsc.subcore_barrier()

    def chunk_body_at(row_base, ci, carry):
        base = row_base + ci * CH
        pltpu.sync_copy(gat_rows.at[pl.ds(base, CH)], gat_c)
        pltpu.sync_copy(sct_rows.at[pl.ds(base, CH)], sct_c)
        pltpu.async_copy(table_hbm.at[gat_c.at[0]], rows0, sem0)

        def body(t, carry2):
            j0 = 2 * t
            pltpu.make_async_copy(table_hbm.at[gat_c.at[j0]], rows0, sem0).wait()
            pltpu.async_copy(table_hbm.at[gat_c.at[j0 + 1]], rows1, sem1)
            pltpu.sync_copy(rows0, acc_sh.at[sct_c.at[j0]], add=True)
            pltpu.make_async_copy(table_hbm.at[gat_c.at[j0 + 1]], rows1, sem1).wait()

            @pl.when(j0 + 2 < CH)
            def _():
                pltpu.async_copy(
                    table_hbm.at[gat_c.at[jnp.minimum(j0 + 2, CH - 1)]], rows0, sem0
                )

            pltpu.sync_copy(rows1, acc_sh.at[sct_c.at[j0 + 1]], add=True)
            return carry2

        lax.fori_loop(0, CH // 2, body, 0)
        return carry

    @pl.when(c == 0)
    def _():
        lax.fori_loop(
            0, RPW0 // CH, lambda ci, cr: chunk_body_at(s * RPW0, ci, cr), 0
        )

    @pl.when(c == 1)
    def _():
        lax.fori_loop(
            0, RPW1 // CH, lambda ci, cr: chunk_body_at(OFF1 + s * RPW1, ci, cr), 0
        )

    plsc.subcore_barrier()
    pltpu.sync_copy(acc_sh.at[pl.ds(s * RPT, RPT)], out.at[c, pl.ds(s * RPT, RPT)])


_agg_sc = pl.kernel(
    _agg_body,
    mesh=_mesh,
    out_type=jax.ShapeDtypeStruct((NC, NP, D), jnp.float32),
    scratch_types=[
        pltpu.VMEM((CH, BA), jnp.int32),
        pltpu.VMEM((CH, BA), jnp.int32),
        pltpu.VMEM((BA, D), jnp.float32),
        pltpu.VMEM((BA, D), jnp.float32),
        pltpu.VMEM_SHARED((NP, D), jnp.float32),
        pltpu.SemaphoreType.DMA,
        pltpu.SemaphoreType.DMA,
    ],
)


def _h_body(hist_ref, feat_ref, h_ref):
    deg = jnp.sum(hist_ref[...], axis=0)[:, None]
    norm = lax.rsqrt(jnp.maximum(deg, 1.0))
    h_ref[...] = feat_ref[...] * norm


def _out_body(hist_ref, p_ref, wc_ref, bc_ref, wl_ref, bl_ref, o_ref):
    deg = jnp.sum(hist_ref[...], axis=0)[:, None]
    norm = lax.rsqrt(jnp.maximum(deg, 1.0))
    a = (p_ref[0] + p_ref[1]) * norm
    h1 = jnp.dot(a, wc_ref[...], preferred_element_type=jnp.float32) + bc_ref[...]
    h1 = jnp.maximum(h1, 0.0)
    h2 = jnp.dot(h1, wl_ref[...], preferred_element_type=jnp.float32) + bl_ref[...]
    o_ref[...] = jnp.maximum(h2, 0.0)


_BN = 512


def kernel(feat, edge_index, W_conv, b_conv, W_lin, b_lin):
    src = edge_index[0].astype(jnp.int32)
    dst = edge_index[1].astype(jnp.int32)
    pad = E_PAD - E
    src_p = jnp.concatenate([src, jnp.zeros((pad,), jnp.int32)])
    dst_p = jnp.concatenate([dst, jnp.full((pad,), N, jnp.int32)])
    src_rows = src_p.reshape(-1, BA)
    dst_rows = dst_p.reshape(-1, BA)
    feat_p = jnp.concatenate([feat, jnp.zeros((NP - N, D), jnp.float32)])
    z128 = jnp.zeros((NP, D), jnp.float32)

    hist = _hist_sc(dst_p.reshape(-1, HB))

    h = pl.pallas_call(
        _h_body,
        grid=(NP // _BN,),
        in_specs=[
            pl.BlockSpec((NWORK, _BN), lambda i: (0, i)),
            pl.BlockSpec((_BN, D), lambda i: (i, 0)),
        ],
        out_specs=pl.BlockSpec((_BN, D), lambda i: (i, 0)),
        out_shape=jax.ShapeDtypeStruct((NP, D), jnp.float32),
    )(hist, feat_p)

    agg_p = _agg_sc(src_rows, dst_rows, h, z128)

    out = pl.pallas_call(
        _out_body,
        grid=(NP // _BN,),
        in_specs=[
            pl.BlockSpec((NWORK, _BN), lambda i: (0, i)),
            pl.BlockSpec((NC, _BN, D), lambda i: (0, i, 0)),
            pl.BlockSpec((D, D), lambda i: (0, 0)),
            pl.BlockSpec((1, D), lambda i: (0, 0)),
            pl.BlockSpec((D, D), lambda i: (0, 0)),
            pl.BlockSpec((1, D), lambda i: (0, 0)),
        ],
        out_specs=pl.BlockSpec((_BN, D), lambda i: (i, 0)),
        out_shape=jax.ShapeDtypeStruct((NP, D), jnp.float32),
    )(hist, agg_p, W_conv.T, b_conv.reshape(1, D), W_lin.T, b_lin.reshape(1, D))

    return out[:N]


# speedup vs baseline: 1.0498x; 1.0050x over previous
import jax
import jax.numpy as jnp
from jax import lax
from jax.experimental import pallas as pl
from jax.experimental.pallas import tpu as pltpu
from jax.experimental.pallas import tpu_sc as plsc

N = 10000
E = 320000
D = 128
NC = 2
NS = 16
NWORK = NC * NS
BA = 128
CH = 16
TOT_ROWS = 2560
RPW0 = 144
RPW1 = 16
OFF1 = NS * RPW0
HB = 64
HROWS = 160
HCH = 16
E_PAD = TOT_ROWS * BA
NP = 10240
RPT = NP // NS

_mesh = plsc.VectorSubcoreMesh(
    core_axis_name="c", subcore_axis_name="s", num_cores=NC, num_subcores=NS
)


def _hist_body(dst_rows, out, dst_v, hist):
    c = lax.axis_index("c")
    s = lax.axis_index("s")
    w = s * NC + c

    def zbody(i, carry):
        hist[pl.ds(i * 16, 16)] = jnp.zeros((16,), jnp.float32)
        return carry

    lax.fori_loop(0, NP // 16, zbody, 0)
    ones = jnp.ones((16,), jnp.float32)

    def chunk(ci, carry):
        pltpu.sync_copy(dst_rows.at[pl.ds(w * HROWS + ci * HCH, HCH)], dst_v)

        def body(j, carry2):
            def inner(kk, carry3):
                idx = dst_v[j, pl.ds(kk * 16, 16)]
                plsc.addupdate_scatter(hist, (idx,), ones)
                return carry3

            lax.fori_loop(0, HB // 16, inner, 0)
            return carry2

        lax.fori_loop(0, HCH, body, 0)
        return carry

    lax.fori_loop(0, HROWS // HCH, chunk, 0)
    pltpu.sync_copy(hist, out.at[w])


_hist_sc = pl.kernel(
    _hist_body,
    mesh=_mesh,
    out_type=jax.ShapeDtypeStruct((NWORK, NP), jnp.float32),
    scratch_types=[
        pltpu.VMEM((HCH, HB), jnp.int32),
        pltpu.VMEM((NP,), jnp.float32),
    ],
    compiler_params=pltpu.CompilerParams(needs_layout_passes=False),
)


def _agg_body(gat_rows, sct_rows, table_hbm, z_hbm, out, gat_c, sct_c,
              rows0, rows1, acc_sh, sem0, sem1):
    c = lax.axis_index("c")
    s = lax.axis_index("s")
    pltpu.sync_copy(z_hbm.at[pl.ds(s * RPT, RPT)], acc_sh.at[pl.ds(s * RPT, RPT)])
    plsc.subcore_barrier()

    def chunk_body_at(row_base, ci, carry):
        base = row_base + ci * CH
        pltpu.sync_copy(gat_rows.at[pl.ds(base, CH)], gat_c)
        pltpu.sync_copy(sct_rows.at[pl.ds(base, CH)], sct_c)
        pltpu.async_copy(table_hbm.at[gat_c.at[0]], rows0, sem0)

        def body(t, carry2):
            j0 = 2 * t
            pltpu.make_async_copy(table_hbm.at[gat_c.at[j0]], rows0, sem0).wait()
            pltpu.async_copy(table_hbm.at[gat_c.at[j0 + 1]], rows1, sem1)
            pltpu.sync_copy(rows0, acc_sh.at[sct_c.at[j0]], add=True)
            pltpu.make_async_copy(table_hbm.at[gat_c.at[j0 + 1]], rows1, sem1).wait()

            @pl.when(j0 + 2 < CH)
            def _():
                pltpu.async_copy(
                    table_hbm.at[gat_c.at[jnp.minimum(j0 + 2, CH - 1)]], rows0, sem0
                )

            pltpu.sync_copy(rows1, acc_sh.at[sct_c.at[j0 + 1]], add=True)
            return carry2

        lax.fori_loop(0, CH // 2, body, 0)
        return carry

    @pl.when(c == 0)
    def _():
        lax.fori_loop(
            0, RPW0 // CH, lambda ci, cr: chunk_body_at(s * RPW0, ci, cr), 0
        )

    @pl.when(c == 1)
    def _():
        lax.fori_loop(
            0, RPW1 // CH, lambda ci, cr: chunk_body_at(OFF1 + s * RPW1, ci, cr), 0
        )

    plsc.subcore_barrier()
    pltpu.sync_copy(acc_sh.at[pl.ds(s * RPT, RPT)], out.at[c, pl.ds(s * RPT, RPT)])


_agg_sc = pl.kernel(
    _agg_body,
    mesh=_mesh,
    out_type=jax.ShapeDtypeStruct((NC, NP, D), jnp.float32),
    scratch_types=[
        pltpu.VMEM((CH, BA), jnp.int32),
        pltpu.VMEM((CH, BA), jnp.int32),
        pltpu.VMEM((BA, D), jnp.float32),
        pltpu.VMEM((BA, D), jnp.float32),
        pltpu.VMEM_SHARED((NP, D), jnp.float32),
        pltpu.SemaphoreType.DMA,
        pltpu.SemaphoreType.DMA,
    ],
)


def _h_body(hist_ref, feat_ref, h_ref):
    deg = jnp.sum(hist_ref[...], axis=0)[:, None]
    norm = lax.rsqrt(jnp.maximum(deg, 1.0))
    h_ref[...] = feat_ref[...] * norm


def _out_body(hist_ref, p_ref, wc_ref, bc_ref, wl_ref, bl_ref, o_ref):
    deg = jnp.sum(hist_ref[...], axis=0)[:, None]
    norm = lax.rsqrt(jnp.maximum(deg, 1.0))
    a = (p_ref[0] + p_ref[1]) * norm
    h1 = jnp.dot(a, wc_ref[...], preferred_element_type=jnp.float32) + bc_ref[...]
    h1 = jnp.maximum(h1, 0.0)
    h2 = jnp.dot(h1, wl_ref[...], preferred_element_type=jnp.float32) + bl_ref[...]
    o_ref[...] = jnp.maximum(h2, 0.0)


_BN = 512


def kernel(feat, edge_index, W_conv, b_conv, W_lin, b_lin):
    src = edge_index[0].astype(jnp.int32)
    dst = edge_index[1].astype(jnp.int32)
    pad = E_PAD - E
    src_p = jnp.concatenate([src, jnp.zeros((pad,), jnp.int32)])
    dst_p = jnp.concatenate([dst, jnp.full((pad,), N, jnp.int32)])
    src_rows = src_p.reshape(-1, BA)
    dst_rows = dst_p.reshape(-1, BA)
    feat_p = jnp.concatenate([feat, jnp.zeros((NP - N, D), jnp.float32)])
    z128 = jnp.zeros((NP, D), jnp.float32)

    hist = _hist_sc(dst_p.reshape(-1, HB))

    h = pl.pallas_call(
        _h_body,
        grid=(NP // _BN,),
        in_specs=[
            pl.BlockSpec((NWORK, _BN), lambda i: (0, i)),
            pl.BlockSpec((_BN, D), lambda i: (i, 0)),
        ],
        out_specs=pl.BlockSpec((_BN, D), lambda i: (i, 0)),
        out_shape=jax.ShapeDtypeStruct((NP, D), jnp.float32),
    )(hist, feat_p)

    agg_p = _agg_sc(src_rows, dst_rows, h, z128)

    out = pl.pallas_call(
        _out_body,
        grid=(NP // _BN,),
        in_specs=[
            pl.BlockSpec((NWORK, _BN), lambda i: (0, i)),
            pl.BlockSpec((NC, _BN, D), lambda i: (0, i, 0)),
            pl.BlockSpec((D, D), lambda i: (0, 0)),
            pl.BlockSpec((1, D), lambda i: (0, 0)),
            pl.BlockSpec((D, D), lambda i: (0, 0)),
            pl.BlockSpec((1, D), lambda i: (0, 0)),
        ],
        out_specs=pl.BlockSpec((_BN, D), lambda i: (i, 0)),
        out_shape=jax.ShapeDtypeStruct((NP, D), jnp.float32),
    )(hist, agg_p, W_conv.T, b_conv.reshape(1, D), W_lin.T, b_lin.reshape(1, D))

    return out[:N]
